# Initial kernel scaffold; baseline (speedup 1.0000x reference)
import jax
import jax.numpy as jnp
from jax.experimental import pallas as pl


def kernel(x, W_A, keys_A, vals_A, W_B, keys_B, vals_B):
    raise NotImplementedError("write your pallas kernel here")


# speedup vs baseline: 9.3891x; 9.3891x over previous
import functools

import numpy as np
import jax
import jax.numpy as jnp
from jax.experimental import pallas as pl

_B, _N, _D, _R = 1, 2048, 1024, 8
_CB, _CIC, _KV = 64, 16, 64
_OP = (_D * _R) // _CB
_G = 8
_T = 256

_PERM = np.arange(_CB).reshape(8, 8).T.reshape(-1)


def _prep(W, keys, vals):
    Wp = W[_PERM].reshape(_CB * _CIC, _D).T
    kp = keys[_PERM].reshape(_G, 8, _KV, _CIC)
    eye = jnp.eye(8, dtype=W.dtype)
    K = jnp.einsum('qjkg,ji->qjgik', kp, eye).reshape(_G, 8 * _CIC, 8 * _KV)
    kn = (kp ** 2).sum(-1).reshape(_G, 1, 8 * _KV)
    V = vals[_PERM].reshape(_G, 8 * _KV, _OP)
    return Wp, K, kn, V


def _body(x_ref, wa_ref, ka_ref, kna_ref, vat_ref,
          wb_ref, kb_ref, knb_ref, vb_ref, out_ref):
    xb = x_ref[...]
    pA = jnp.dot(xb, wa_ref[...], preferred_element_type=jnp.float32)
    pB = jnp.dot(xb, wb_ref[...], preferred_element_type=jnp.float32)
    io = jax.lax.broadcasted_iota(jnp.int32, (_T, _KV), 1)
    t = [None] * 8
    kminB = [[None] * 8 for _ in range(_G)]
    for q in range(_G):
        xq = xb[:, q * 128:(q + 1) * 128]
        crossA = jnp.dot(pA[:, q * 128:(q + 1) * 128], ka_ref[q],
                         preferred_element_type=jnp.float32)
        sA = kna_ref[q] - 2.0 * crossA
        P = jnp.dot(xq, vat_ref[q], preferred_element_type=jnp.float32)
        crossB = jnp.dot(pB[:, q * 128:(q + 1) * 128], kb_ref[q],
                         preferred_element_type=jnp.float32)
        sB = knb_ref[q] - 2.0 * crossB
        for j in range(8):
            vA = sA[:, j * 64:(j + 1) * 64]
            mA = jnp.min(vA, axis=1, keepdims=True)
            kA = jnp.min(jnp.where(vA == mA, io, _KV), axis=1, keepdims=True)
            ohA = (io == kA).astype(jnp.float32)
            s = jnp.sum(P[:, j * 64:(j + 1) * 64] * ohA, axis=1, keepdims=True)
            t[j] = s if t[j] is None else t[j] + s
            vB = sB[:, j * 64:(j + 1) * 64]
            mB = jnp.min(vB, axis=1, keepdims=True)
            kminB[q][j] = jnp.min(jnp.where(vB == mB, io, _KV), axis=1,
                                  keepdims=True)
    for q in range(_G):
        w = jnp.concatenate(
            [(io == kminB[q][j]).astype(jnp.float32) * t[j] for j in range(8)],
            axis=1)
        out_ref[:, q * 128:(q + 1) * 128] = jnp.dot(
            w, vb_ref[q], preferred_element_type=jnp.float32)


@functools.partial(jax.jit, static_argnames=("interpret",))
def _run(x, W_A, keys_A, vals_A, W_B, keys_B, vals_B, interpret=False):
    WAp, KA, knA, VA = _prep(W_A, keys_A, vals_A)
    WBp, KB, knB, VB = _prep(W_B, keys_B, vals_B)
    VAT = VA.transpose(0, 2, 1)
    full = lambda *s: pl.BlockSpec(s, lambda i: (0,) * len(s))
    out = pl.pallas_call(
        _body,
        grid=(_N // _T,),
        in_specs=[
            pl.BlockSpec((_T, _D), lambda i: (i, 0)),
            full(_D, _CB * _CIC),
            full(_G, 8 * _CIC, 8 * _KV),
            full(_G, 1, 8 * _KV),
            full(_G, 8 * _CIC, 8 * _KV),
            full(_D, _CB * _CIC),
            full(_G, 8 * _CIC, 8 * _KV),
            full(_G, 1, 8 * _KV),
            full(_G, 8 * _KV, _OP),
        ],
        out_specs=pl.BlockSpec((_T, _D), lambda i: (i, 0)),
        out_shape=jax.ShapeDtypeStruct((_N, _D), jnp.float32),
        interpret=interpret,
    )(x.reshape(_N, _D), WAp, KA, knA, VAT, WBp, KB, knB, VB)
    return out.reshape(_B, _N, _D)


def kernel(x, W_A, keys_A, vals_A, W_B, keys_B, vals_B):
    return _run(x, W_A, keys_A, vals_A, W_B, keys_B, vals_B)


# speedup vs baseline: 30.0084x; 3.1961x over previous
import functools

import numpy as np
import jax
import jax.numpy as jnp
from jax.experimental import pallas as pl

_B, _N, _D, _R = 1, 2048, 1024, 8
_CB, _CIC, _KV = 64, 16, 64
_OP = (_D * _R) // _CB
_G = 8
_T = 256

_PERM = np.arange(_CB).reshape(8, 8).T.reshape(-1)


def _prep(W, keys, vals):
    Wt = W[_PERM].reshape(_CB * _CIC, _D)
    kp = keys[_PERM].reshape(_G, 8, _KV, _CIC)
    eye = jnp.eye(8, dtype=W.dtype)
    KT = jnp.einsum('qjkg,ji->qikjg', kp, eye).reshape(_G, 8 * _KV, 8 * _CIC)
    kn = (kp ** 2).sum(-1).reshape(_G, 8 * _KV, 1)
    V = vals[_PERM].reshape(_G, 8 * _KV, _OP)
    return Wt, KT, kn, V


def _kmin_oh(sc, ko):
    m = jnp.min(sc, axis=1, keepdims=True)
    kmin = jnp.min(jnp.where(sc == m, ko, _KV), axis=1, keepdims=True)
    return kmin


def _body(xt_ref, wa_ref, ka_ref, kna_ref, va_ref,
          wb_ref, kb_ref, knb_ref, vbt_ref, out_ref):
    xt = xt_ref[...]
    pTA = jnp.dot(wa_ref[...], xt, preferred_element_type=jnp.float32)
    pTB = jnp.dot(wb_ref[...], xt, preferred_element_type=jnp.float32)
    ko = jax.lax.broadcasted_iota(jnp.int32, (8, _KV, _T), 1)
    t = None
    kminB = [None] * _G
    for q in range(_G):
        xq = xt[q * 128:(q + 1) * 128, :]
        crossA = jnp.dot(ka_ref[q], pTA[q * 128:(q + 1) * 128, :],
                         preferred_element_type=jnp.float32)
        scA = (kna_ref[q] - 2.0 * crossA).reshape(8, _KV, _T)
        kA = _kmin_oh(scA, ko)
        PT = jnp.dot(va_ref[q], xq,
                     preferred_element_type=jnp.float32).reshape(8, _KV, _T)
        s = jnp.sum(jnp.where(ko == kA, PT, 0.0), axis=1, keepdims=True)
        t = s if t is None else t + s
        crossB = jnp.dot(kb_ref[q], pTB[q * 128:(q + 1) * 128, :],
                         preferred_element_type=jnp.float32)
        scB = (knb_ref[q] - 2.0 * crossB).reshape(8, _KV, _T)
        kminB[q] = _kmin_oh(scB, ko)
    for q in range(_G):
        w = jnp.where(ko == kminB[q], jnp.broadcast_to(t, ko.shape), 0.0)
        out_ref[q * 128:(q + 1) * 128, :] = jnp.dot(
            vbt_ref[q], w.reshape(8 * _KV, _T),
            preferred_element_type=jnp.float32)


@functools.partial(jax.jit, static_argnames=("interpret",))
def _run(x, W_A, keys_A, vals_A, W_B, keys_B, vals_B, interpret=False):
    WAt, KAT, knA, VA = _prep(W_A, keys_A, vals_A)
    WBt, KBT, knB, VB = _prep(W_B, keys_B, vals_B)
    VBT = VB.transpose(0, 2, 1)
    xt = x.reshape(_N, _D).T
    full = lambda *s: pl.BlockSpec(s, lambda i: (0,) * len(s))
    outT = pl.pallas_call(
        _body,
        grid=(_N // _T,),
        in_specs=[
            pl.BlockSpec((_D, _T), lambda i: (0, i)),
            full(_CB * _CIC, _D),
            full(_G, 8 * _KV, 8 * _CIC),
            full(_G, 8 * _KV, 1),
            full(_G, 8 * _KV, _OP),
            full(_CB * _CIC, _D),
            full(_G, 8 * _KV, 8 * _CIC),
            full(_G, 8 * _KV, 1),
            full(_G, _OP, 8 * _KV),
        ],
        out_specs=pl.BlockSpec((_D, _T), lambda i: (0, i)),
        out_shape=jax.ShapeDtypeStruct((_D, _N), jnp.float32),
        interpret=interpret,
    )(xt, WAt, KAT, knA, VA, WBt, KBT, knB, VBT)
    return outT.T.reshape(_B, _N, _D)


def kernel(x, W_A, keys_A, vals_A, W_B, keys_B, vals_B):
    return _run(x, W_A, keys_A, vals_A, W_B, keys_B, vals_B)
